# Initial kernel scaffold; baseline (speedup 1.0000x reference)
import functools

import jax
import jax.numpy as jnp
from jax import lax
from jax.experimental import pallas as pl
from jax.experimental.pallas import tpu as pltpu
from jax.experimental.pallas import tpu_sc as plsc

NC = 2
NS = 16
NW = NC * NS
CH = 1024


@functools.partial(jax.jit, static_argnums=(2, 3))
def _emb_lookup(ids, table, n_per_w, d):
    n_ch = n_per_w // CH
    mesh = plsc.VectorSubcoreMesh(
        core_axis_name="c", subcore_axis_name="s",
        num_cores=NC, num_subcores=NS)

    @functools.partial(
        pl.kernel,
        out_type=jax.ShapeDtypeStruct((n_per_w * NW, d), jnp.float32),
        mesh=mesh,
        scratch_types=[
            pltpu.VMEM((CH,), jnp.int32),
            pltpu.VMEM((CH, d), jnp.float32),
            pltpu.SemaphoreType.DMA,
        ],
    )
    def k(ids_hbm, table_hbm, out_hbm, idx_v, rows_v, sem):
        wid = lax.axis_index("s") * NC + lax.axis_index("c")
        base = wid * n_per_w

        def step(i, carry):
            off = base + i * CH
            pltpu.sync_copy(ids_hbm.at[pl.ds(off, CH)], idx_v)
            pltpu.async_copy(table_hbm.at[idx_v], rows_v, sem).wait()
            pltpu.sync_copy(rows_v, out_hbm.at[pl.ds(off, CH)])
            return carry

        lax.fori_loop(0, n_ch, step, 0)

    return k(ids, table)


def kernel(token_ids, table):
    b, s = token_ids.shape
    d = table.shape[1]
    n = b * s
    ids = token_ids.reshape(n).astype(jnp.int32)
    out = _emb_lookup(ids, table, n // NW, d)
    return out.reshape(b, s, d)


# speedup vs baseline: 1.4577x; 1.4577x over previous
import functools

import jax
import jax.numpy as jnp
from jax import lax
from jax.experimental import pallas as pl
from jax.experimental.pallas import tpu as pltpu
from jax.experimental.pallas import tpu_sc as plsc

NC = 2
NS = 16
NW = NC * NS
CH = 1024


@functools.partial(jax.jit, static_argnums=(2, 3))
def _emb_lookup(ids, table, n_per_w, d):
    n_ch = n_per_w // CH
    mesh = plsc.VectorSubcoreMesh(
        core_axis_name="c", subcore_axis_name="s",
        num_cores=NC, num_subcores=NS)

    @functools.partial(
        pl.kernel,
        out_type=jax.ShapeDtypeStruct((n_per_w * NW, d), jnp.float32),
        mesh=mesh,
        scratch_types=[
            pltpu.VMEM((CH,), jnp.int32),
            pltpu.VMEM((CH, d), jnp.float32),
            pltpu.SemaphoreType.DMA,
        ],
        compiler_params=pltpu.CompilerParams(use_tc_tiling_on_sc=False),
    )
    def k(ids_hbm, table_hbm, out_hbm, idx_v, rows_v, sem):
        wid = lax.axis_index("s") * NC + lax.axis_index("c")
        base = wid * n_per_w

        def step(i, carry):
            off = base + i * CH
            pltpu.sync_copy(ids_hbm.at[pl.ds(off, CH)], idx_v)
            pltpu.async_copy(table_hbm.at[idx_v], rows_v, sem).wait()
            pltpu.sync_copy(rows_v, out_hbm.at[pl.ds(off, CH)])
            return carry

        lax.fori_loop(0, n_ch, step, 0)

    return k(ids, table)


def kernel(token_ids, table):
    b, s = token_ids.shape
    d = table.shape[1]
    n = b * s
    ids = token_ids.reshape(n).astype(jnp.int32)
    out = _emb_lookup(ids, table, n // NW, d)
    return out.reshape(b, s, d)


# speedup vs baseline: 1.4986x; 1.0280x over previous
import functools

import jax
import jax.numpy as jnp
from jax import lax
from jax.experimental import pallas as pl
from jax.experimental.pallas import tpu as pltpu
from jax.experimental.pallas import tpu_sc as plsc

NC = 2
NS = 16
NW = NC * NS
CH = 1024
NBUF = 3


@functools.partial(jax.jit, static_argnums=(2, 3))
def _emb_lookup(ids, table, n_per_w, d):
    n_ch = n_per_w // CH
    mesh = plsc.VectorSubcoreMesh(
        core_axis_name="c", subcore_axis_name="s",
        num_cores=NC, num_subcores=NS)

    @functools.partial(
        pl.kernel,
        out_type=jax.ShapeDtypeStruct((n_per_w * NW, d), jnp.float32),
        mesh=mesh,
        scratch_types=[
            [pltpu.VMEM((CH,), jnp.int32) for _ in range(NBUF)],
            [pltpu.VMEM((CH, d), jnp.float32) for _ in range(NBUF)],
            [pltpu.SemaphoreType.DMA for _ in range(NBUF)],
            [pltpu.SemaphoreType.DMA for _ in range(NBUF)],
            [pltpu.SemaphoreType.DMA for _ in range(NBUF)],
        ],
        compiler_params=pltpu.CompilerParams(use_tc_tiling_on_sc=False),
    )
    def k(ids_hbm, table_hbm, out_hbm, idx_v, rows_v, sem_i, sem_g, sem_s):
        wid = lax.axis_index("s") * NC + lax.axis_index("c")
        base = wid * n_per_w

        def off(i):
            return base + i * CH

        idx_h = [None] * n_ch
        gather_h = [None] * n_ch
        store_h = [None] * n_ch

        for i in range(min(NBUF, n_ch)):
            idx_h[i] = pltpu.async_copy(
                ids_hbm.at[pl.ds(off(i), CH)], idx_v[i], sem_i[i])

        for i in range(n_ch):
            b = i % NBUF
            if i >= NBUF:
                store_h[i - NBUF].wait()
            idx_h[i].wait()
            gather_h[i] = pltpu.async_copy(
                table_hbm.at[idx_v[b]], rows_v[b], sem_g[b])
            j = i - (NBUF - 1)
            if j >= 0:
                bj = j % NBUF
                gather_h[j].wait()
                if j + NBUF < n_ch:
                    idx_h[j + NBUF] = pltpu.async_copy(
                        ids_hbm.at[pl.ds(off(j + NBUF), CH)],
                        idx_v[bj], sem_i[bj])
                store_h[j] = pltpu.async_copy(
                    rows_v[bj], out_hbm.at[pl.ds(off(j), CH)], sem_s[bj])

        for j in range(max(0, n_ch - (NBUF - 1)), n_ch):
            bj = j % NBUF
            gather_h[j].wait()
            store_h[j] = pltpu.async_copy(
                rows_v[bj], out_hbm.at[pl.ds(off(j), CH)], sem_s[bj])

        for j in range(max(0, n_ch - NBUF), n_ch):
            store_h[j].wait()

    return k(ids, table)


def kernel(token_ids, table):
    b, s = token_ids.shape
    d = table.shape[1]
    n = b * s
    ids = token_ids.reshape(n).astype(jnp.int32)
    out = _emb_lookup(ids, table, n // NW, d)
    return out.reshape(b, s, d)
